# baseline (speedup 1.0000x reference)
import functools

import jax
import jax.numpy as jnp
from jax import lax
from jax.experimental import pallas as pl
from jax.experimental.pallas import tpu as pltpu
from jax.experimental.pallas import tpu_sc as plsc

V = 8192
N_TOK = 256 * 128


_ROWS_PER_BLK = 128
_N_BLKS = V // _ROWS_PER_BLK


def _lse_body(x_ref, o_ref, p_ref):
    x = x_ref[...]
    m = jnp.max(x, axis=1)
    s = jnp.sum(jnp.exp(x - m[:, None]), axis=1)
    o_ref[...] = (m + jnp.log(s)).reshape(1, 1, _ROWS_PER_BLK)
    u = lax.bitcast_convert_type(x, jnp.int32)
    r16 = lax.shift_right_logical(u + 0x8000, 16)
    word = r16[:, : V // 2] | lax.shift_left(r16[:, V // 2 :], 16)
    p_ref[...] = word.reshape(_ROWS_PER_BLK, V // 256, 128)


def _row_logsumexp(table):
    logz, packed = pl.pallas_call(
        _lse_body,
        grid=(_N_BLKS,),
        in_specs=[pl.BlockSpec((_ROWS_PER_BLK, V), lambda i: (i, 0))],
        out_specs=[
            pl.BlockSpec((1, 1, _ROWS_PER_BLK), lambda i: (i, 0, 0)),
            pl.BlockSpec((_ROWS_PER_BLK, V // 256, 128), lambda i: (i, 0, 0)),
        ],
        out_shape=[
            jax.ShapeDtypeStruct((_N_BLKS, 1, _ROWS_PER_BLK), jnp.float32),
            jax.ShapeDtypeStruct((V, V // 256, 128), jnp.int32),
        ],
    )(table)
    return logz.reshape(V), packed.reshape(V * V // 2)


_NC, _NS, _L = 2, 16, 16
_NW = _NC * _NS
_BPW = N_TOK // _NW
_CH = 128
_NCH = _BPW // _CH

_sc_mesh = plsc.VectorSubcoreMesh(core_axis_name="c", subcore_axis_name="s")


@functools.partial(
    pl.kernel,
    out_type=jax.ShapeDtypeStruct((_NW * _L,), jnp.float32),
    mesh=_sc_mesh,
    scratch_types=[
        pltpu.VMEM((_NCH, _CH), jnp.int32),
        pltpu.VMEM((_NCH, _CH), jnp.int32),
        pltpu.VMEM((_BPW,), jnp.int32),
        pltpu.VMEM((_BPW,), jnp.int32),
        pltpu.VMEM((_BPW,), jnp.float32),
        pltpu.VMEM((_L,), jnp.float32),
        pltpu.SemaphoreType.DMA,
    ],
)
def _sc_gather(word_idx_hbm, blocks_hbm, par_hbm, packed_hbm, logz_hbm,
               out_hbm, idx_v, blk_v, par_v, vals_v, lz_v, part_v, sem):
    wid = lax.axis_index("s") * _NC + lax.axis_index("c")

    pltpu.sync_copy(word_idx_hbm.at[wid], idx_v)
    pltpu.sync_copy(blocks_hbm.at[wid], blk_v)
    copies = []
    for j in range(_NCH):
        copies.append(
            pltpu.async_copy(packed_hbm.at[idx_v.at[j]],
                             vals_v.at[pl.ds(j * _CH, _CH)], sem))
        copies.append(
            pltpu.async_copy(logz_hbm.at[blk_v.at[j]],
                             lz_v.at[pl.ds(j * _CH, _CH)], sem))
    pltpu.sync_copy(par_hbm.at[wid], par_v)
    for cp in copies:
        cp.wait()

    def body(i, acc):
        w = vals_v[pl.ds(i * _L, _L)]
        p = par_v[pl.ds(i * _L, _L)]
        bits = lax.shift_right_logical(w, p * 16) & 0xFFFF
        s = lax.shift_right_logical(bits, 15)
        e = lax.shift_right_logical(bits, 7) & 0xFF
        m = bits & 0x7F
        sign = 1.0 - 2.0 * s.astype(jnp.float32)
        frac = 1.0 + m.astype(jnp.float32) * (1.0 / 128.0)
        mag = jnp.exp((e.astype(jnp.float32) - 127.0) * 0.6931471805599453)
        tv = sign * frac * mag
        lz = lz_v[pl.ds(i * _L, _L)]
        return acc + (lz - tv)

    acc = lax.fori_loop(0, _BPW // _L, body, jnp.zeros((_L,), jnp.float32))
    part_v[...] = acc
    pltpu.sync_copy(part_v, out_hbm.at[pl.ds(wid * _L, _L)])


def kernel(blocks, targets, table):
    b = blocks.reshape(-1).astype(jnp.int32)
    t = targets.reshape(-1).astype(jnp.int32)
    word_idx = (b * (V // 2) + (t & (V // 2 - 1))).reshape(_NW, _NCH, _CH)
    parity = lax.shift_right_logical(t, 12).reshape(_NW, _BPW)
    b_sh = b.reshape(_NW, _NCH, _CH)
    logz, packed = _row_logsumexp(table)
    parts = _sc_gather(word_idx, b_sh, parity, packed, logz)
    return jnp.sum(parts) / N_TOK


# speedup vs baseline: 1.1284x; 1.1284x over previous
import functools

import jax
import jax.numpy as jnp
from jax import lax
from jax.experimental import pallas as pl
from jax.experimental.pallas import tpu as pltpu
from jax.experimental.pallas import tpu_sc as plsc

V = 8192
N_TOK = 256 * 128


_ROWS_PER_BLK = 256
_N_BLKS = V // _ROWS_PER_BLK


def _lse_body(x_ref, o_ref, p_ref):
    x = x_ref[...]
    m = jnp.max(x, axis=1)
    s = jnp.sum(jnp.exp(x - m[:, None]), axis=1)
    o_ref[...] = (m + jnp.log(s)).reshape(1, 1, _ROWS_PER_BLK)
    u = lax.bitcast_convert_type(x, jnp.int32) + 0x8000
    word = (lax.shift_right_logical(u[:, : V // 2], 16)
            | (u[:, V // 2 :] & -0x10000))
    p_ref[...] = word.reshape(_ROWS_PER_BLK, V // 256, 128)


def _row_logsumexp(table):
    logz, packed = pl.pallas_call(
        _lse_body,
        grid=(_N_BLKS,),
        in_specs=[pl.BlockSpec((_ROWS_PER_BLK, V), lambda i: (i, 0))],
        out_specs=[
            pl.BlockSpec((1, 1, _ROWS_PER_BLK), lambda i: (i, 0, 0)),
            pl.BlockSpec((_ROWS_PER_BLK, V // 256, 128), lambda i: (i, 0, 0)),
        ],
        out_shape=[
            jax.ShapeDtypeStruct((_N_BLKS, 1, _ROWS_PER_BLK), jnp.float32),
            jax.ShapeDtypeStruct((V, V // 256, 128), jnp.int32),
        ],
    )(table)
    return logz.reshape(V), packed.reshape(V * V // 2)


_NC, _NS, _L = 2, 16, 16
_NW = _NC * _NS
_BPW = N_TOK // _NW
_CH = 128
_NCH = _BPW // _CH

_sc_mesh = plsc.VectorSubcoreMesh(core_axis_name="c", subcore_axis_name="s")


@functools.partial(
    pl.kernel,
    out_type=jax.ShapeDtypeStruct((_NW * _L,), jnp.float32),
    mesh=_sc_mesh,
    scratch_types=[
        pltpu.VMEM((_NCH, _CH), jnp.int32),
        pltpu.VMEM((_NCH, _CH), jnp.int32),
        pltpu.VMEM((_BPW,), jnp.int32),
        pltpu.VMEM((_BPW,), jnp.int32),
        pltpu.VMEM((_BPW,), jnp.float32),
        pltpu.VMEM((_L,), jnp.float32),
        pltpu.SemaphoreType.DMA,
    ],
)
def _sc_gather(word_idx_hbm, blocks_hbm, par_hbm, packed_hbm, logz_hbm,
               out_hbm, idx_v, blk_v, par_v, vals_v, lz_v, part_v, sem):
    wid = lax.axis_index("s") * _NC + lax.axis_index("c")

    pltpu.sync_copy(word_idx_hbm.at[wid], idx_v)
    pltpu.sync_copy(blocks_hbm.at[wid], blk_v)
    copies = []
    for j in range(_NCH):
        copies.append(
            pltpu.async_copy(packed_hbm.at[idx_v.at[j]],
                             vals_v.at[pl.ds(j * _CH, _CH)], sem))
        copies.append(
            pltpu.async_copy(logz_hbm.at[blk_v.at[j]],
                             lz_v.at[pl.ds(j * _CH, _CH)], sem))
    pltpu.sync_copy(par_hbm.at[wid], par_v)
    for cp in copies:
        cp.wait()

    def body(i, acc):
        w = vals_v[pl.ds(i * _L, _L)]
        p = par_v[pl.ds(i * _L, _L)]
        bits = lax.shift_right_logical(w, p * 16) & 0xFFFF
        s = lax.shift_right_logical(bits, 15)
        e = lax.shift_right_logical(bits, 7) & 0xFF
        m = bits & 0x7F
        sign = 1.0 - 2.0 * s.astype(jnp.float32)
        frac = 1.0 + m.astype(jnp.float32) * (1.0 / 128.0)
        mag = jnp.exp((e.astype(jnp.float32) - 127.0) * 0.6931471805599453)
        tv = sign * frac * mag
        lz = lz_v[pl.ds(i * _L, _L)]
        return acc + (lz - tv)

    acc = lax.fori_loop(0, _BPW // _L, body, jnp.zeros((_L,), jnp.float32))
    part_v[...] = acc
    pltpu.sync_copy(part_v, out_hbm.at[pl.ds(wid * _L, _L)])


def kernel(blocks, targets, table):
    b = blocks.reshape(-1).astype(jnp.int32)
    t = targets.reshape(-1).astype(jnp.int32)
    word_idx = (b * (V // 2) + (t & (V // 2 - 1))).reshape(_NW, _NCH, _CH)
    parity = lax.shift_right_logical(t, 12).reshape(_NW, _BPW)
    b_sh = b.reshape(_NW, _NCH, _CH)
    logz, packed = _row_logsumexp(table)
    parts = _sc_gather(word_idx, b_sh, parity, packed, logz)
    return jnp.sum(parts) / N_TOK


# speedup vs baseline: 1.1325x; 1.0037x over previous
import functools

import jax
import jax.numpy as jnp
from jax import lax
from jax.experimental import pallas as pl
from jax.experimental.pallas import tpu as pltpu
from jax.experimental.pallas import tpu_sc as plsc

V = 8192
N_TOK = 256 * 128


_ROWS_PER_BLK = 256
_N_BLKS = V // _ROWS_PER_BLK


def _lse_body(x_ref, o_ref, p_ref):
    x = x_ref[...]
    m = jnp.max(x, axis=1)
    s = jnp.sum(jnp.exp(x - m[:, None]), axis=1)
    o_ref[...] = (m + jnp.log(s)).reshape(1, 1, _ROWS_PER_BLK)
    u = lax.bitcast_convert_type(x, jnp.int32) + 0x8000
    word = (lax.shift_right_logical(u[:, : V // 2], 16)
            | (u[:, V // 2 :] & -0x10000))
    p_ref[...] = word.reshape(_ROWS_PER_BLK, V // 256, 128)


def _row_logsumexp(table):
    logz, packed = pl.pallas_call(
        _lse_body,
        grid=(_N_BLKS,),
        in_specs=[pl.BlockSpec((_ROWS_PER_BLK, V), lambda i: (i, 0))],
        out_specs=[
            pl.BlockSpec((1, 1, _ROWS_PER_BLK), lambda i: (i, 0, 0)),
            pl.BlockSpec((_ROWS_PER_BLK, V // 256, 128), lambda i: (i, 0, 0)),
        ],
        out_shape=[
            jax.ShapeDtypeStruct((_N_BLKS, 1, _ROWS_PER_BLK), jnp.float32),
            jax.ShapeDtypeStruct((V, V // 256, 128), jnp.int32),
        ],
    )(table)
    return logz.reshape(V), packed.reshape(V * V // 2)


_NC, _NS, _L = 2, 16, 16
_NW = _NC * _NS
_BPW = N_TOK // _NW
_CH = 128
_NCH = _BPW // _CH
_VPC = _CH // _L

_sc_mesh = plsc.VectorSubcoreMesh(core_axis_name="c", subcore_axis_name="s")


@functools.partial(
    pl.kernel,
    out_type=jax.ShapeDtypeStruct((_NW * _L,), jnp.float32),
    mesh=_sc_mesh,
    scratch_types=[
        pltpu.VMEM((_NCH, _CH), jnp.int32),
        pltpu.VMEM((_NCH, _CH), jnp.int32),
        pltpu.VMEM((_NCH, _CH), jnp.int32),
        pltpu.VMEM((_BPW,), jnp.int32),
        pltpu.VMEM((_BPW,), jnp.float32),
        pltpu.VMEM((_L,), jnp.float32),
        pltpu.SemaphoreType.DMA,
    ],
)
def _sc_gather(blocks_hbm, targets_hbm, packed_hbm, logz_hbm,
               out_hbm, idx_v, blk_v, tgt_v, vals_v, lz_v, part_v, sem):
    wid = lax.axis_index("s") * _NC + lax.axis_index("c")

    pltpu.sync_copy(blocks_hbm.at[wid], blk_v)
    pltpu.sync_copy(targets_hbm.at[wid], tgt_v)
    for j in range(_NCH):
        for k in range(_VPC):
            bb = blk_v[j, pl.ds(k * _L, _L)]
            tt = tgt_v[j, pl.ds(k * _L, _L)]
            idx_v[j, pl.ds(k * _L, _L)] = bb * (V // 2) + (tt & (V // 2 - 1))

    copies = []
    for j in range(_NCH):
        copies.append(
            pltpu.async_copy(packed_hbm.at[idx_v.at[j]],
                             vals_v.at[pl.ds(j * _CH, _CH)], sem))
        copies.append(
            pltpu.async_copy(logz_hbm.at[blk_v.at[j]],
                             lz_v.at[pl.ds(j * _CH, _CH)], sem))
    for cp in copies:
        cp.wait()

    acc = jnp.zeros((_L,), jnp.float32)
    for j in range(_NCH):
        for k in range(_VPC):
            off = j * _CH + k * _L
            w = vals_v[pl.ds(off, _L)]
            p = lax.shift_right_logical(tgt_v[j, pl.ds(k * _L, _L)], 12)
            bits = lax.shift_right_logical(w, p * 16) & 0xFFFF
            s = lax.shift_right_logical(bits, 15)
            e = lax.shift_right_logical(bits, 7) & 0xFF
            m = bits & 0x7F
            sign = 1.0 - 2.0 * s.astype(jnp.float32)
            frac = 1.0 + m.astype(jnp.float32) * (1.0 / 128.0)
            mag = jnp.exp((e.astype(jnp.float32) - 127.0)
                          * 0.6931471805599453)
            tv = sign * frac * mag
            lz = lz_v[pl.ds(off, _L)]
            acc = acc + (lz - tv)

    part_v[...] = acc
    pltpu.sync_copy(part_v, out_hbm.at[pl.ds(wid * _L, _L)])


def kernel(blocks, targets, table):
    b_sh = blocks.astype(jnp.int32).reshape(_NW, _NCH, _CH)
    t_sh = targets.astype(jnp.int32).reshape(_NW, _NCH, _CH)
    logz, packed = _row_logsumexp(table)
    parts = _sc_gather(b_sh, t_sh, packed, logz)
    return jnp.sum(parts) / N_TOK


# speedup vs baseline: 1.1803x; 1.0422x over previous
import functools

import jax
import jax.numpy as jnp
from jax import lax
from jax.experimental import pallas as pl
from jax.experimental.pallas import tpu as pltpu
from jax.experimental.pallas import tpu_sc as plsc

V = 8192
N_TOK = 256 * 128


_ROWS_PER_BLK = 512
_N_BLKS = V // _ROWS_PER_BLK


def _lse_body(x_ref, o_ref, p_ref):
    for h in range(_ROWS_PER_BLK // 256):
        x = x_ref[pl.ds(h * 256, 256), :]
        m = jnp.max(x, axis=1)
        s = jnp.sum(jnp.exp(x - m[:, None]), axis=1)
        o_ref[0, 0, pl.ds(h * 256, 256)] = m + jnp.log(s)
        u = lax.bitcast_convert_type(x, jnp.int32) + 0x8000
        word = (lax.shift_right_logical(u[:, : V // 2], 16)
                | (u[:, V // 2 :] & -0x10000))
        p_ref[pl.ds(h * 256, 256), :, :] = word.reshape(256, V // 256, 128)


def _row_logsumexp(table):
    logz, packed = pl.pallas_call(
        _lse_body,
        grid=(_N_BLKS,),
        in_specs=[pl.BlockSpec((_ROWS_PER_BLK, V), lambda i: (i, 0))],
        out_specs=[
            pl.BlockSpec((1, 1, _ROWS_PER_BLK), lambda i: (i, 0, 0)),
            pl.BlockSpec((_ROWS_PER_BLK, V // 256, 128), lambda i: (i, 0, 0)),
        ],
        out_shape=[
            jax.ShapeDtypeStruct((_N_BLKS, 1, _ROWS_PER_BLK), jnp.float32),
            jax.ShapeDtypeStruct((V, V // 256, 128), jnp.int32),
        ],
    )(table)
    return logz.reshape(V), packed.reshape(V * V // 2)


_NC, _NS, _L = 2, 16, 16
_NW = _NC * _NS
_BPW = N_TOK // _NW
_CH = 128
_NCH = _BPW // _CH
_VPC = _CH // _L

_sc_mesh = plsc.VectorSubcoreMesh(core_axis_name="c", subcore_axis_name="s")


@functools.partial(
    pl.kernel,
    out_type=jax.ShapeDtypeStruct((_NW * _L,), jnp.float32),
    mesh=_sc_mesh,
    scratch_types=[
        pltpu.VMEM((_NCH, _CH), jnp.int32),
        pltpu.VMEM((_NCH, _CH), jnp.int32),
        pltpu.VMEM((_NCH, _CH), jnp.int32),
        pltpu.VMEM((_BPW,), jnp.int32),
        pltpu.VMEM((_BPW,), jnp.float32),
        pltpu.VMEM((_L,), jnp.float32),
        pltpu.SemaphoreType.DMA,
    ],
)
def _sc_gather(blocks_hbm, targets_hbm, packed_hbm, logz_hbm,
               out_hbm, idx_v, blk_v, tgt_v, vals_v, lz_v, part_v, sem):
    wid = lax.axis_index("s") * _NC + lax.axis_index("c")

    pltpu.sync_copy(blocks_hbm.at[wid], blk_v)
    pltpu.sync_copy(targets_hbm.at[wid], tgt_v)
    for j in range(_NCH):
        for k in range(_VPC):
            bb = blk_v[j, pl.ds(k * _L, _L)]
            tt = tgt_v[j, pl.ds(k * _L, _L)]
            idx_v[j, pl.ds(k * _L, _L)] = bb * (V // 2) + (tt & (V // 2 - 1))

    copies = []
    for j in range(_NCH):
        copies.append(
            pltpu.async_copy(packed_hbm.at[idx_v.at[j]],
                             vals_v.at[pl.ds(j * _CH, _CH)], sem))
        copies.append(
            pltpu.async_copy(logz_hbm.at[blk_v.at[j]],
                             lz_v.at[pl.ds(j * _CH, _CH)], sem))
    for cp in copies:
        cp.wait()

    acc = jnp.zeros((_L,), jnp.float32)
    for j in range(_NCH):
        for k in range(_VPC):
            off = j * _CH + k * _L
            w = vals_v[pl.ds(off, _L)]
            p = lax.shift_right_logical(tgt_v[j, pl.ds(k * _L, _L)], 12)
            bits = lax.shift_right_logical(w, p * 16) & 0xFFFF
            s = lax.shift_right_logical(bits, 15)
            e = lax.shift_right_logical(bits, 7) & 0xFF
            m = bits & 0x7F
            sign = 1.0 - 2.0 * s.astype(jnp.float32)
            frac = 1.0 + m.astype(jnp.float32) * (1.0 / 128.0)
            mag = jnp.exp((e.astype(jnp.float32) - 127.0)
                          * 0.6931471805599453)
            tv = sign * frac * mag
            lz = lz_v[pl.ds(off, _L)]
            acc = acc + (lz - tv)

    part_v[...] = acc
    pltpu.sync_copy(part_v, out_hbm.at[pl.ds(wid * _L, _L)])


def kernel(blocks, targets, table):
    b_sh = blocks.astype(jnp.int32).reshape(_NW, _NCH, _CH)
    t_sh = targets.astype(jnp.int32).reshape(_NW, _NCH, _CH)
    logz, packed = _row_logsumexp(table)
    parts = _sc_gather(b_sh, t_sh, packed, logz)
    return jnp.sum(parts) / N_TOK


# speedup vs baseline: 1.1837x; 1.0029x over previous
import functools

import jax
import jax.numpy as jnp
from jax import lax
from jax.experimental import pallas as pl
from jax.experimental.pallas import tpu as pltpu
from jax.experimental.pallas import tpu_sc as plsc

V = 8192
N_TOK = 256 * 128


_ROWS_PER_BLK = 512
_N_BLKS = V // _ROWS_PER_BLK


def _lse_body(x_ref, o_ref, p_ref):
    for h in range(_ROWS_PER_BLK // 128):
        x = x_ref[pl.ds(h * 128, 128), :]
        m = jnp.max(x, axis=1)
        s = jnp.sum(jnp.exp(x - m[:, None]), axis=1)
        o_ref[0, 0, pl.ds(h * 128, 128)] = m + jnp.log(s)
        u = lax.bitcast_convert_type(x, jnp.int32) + 0x8000
        word = (lax.shift_right_logical(u[:, : V // 2], 16)
                | (u[:, V // 2 :] & -0x10000))
        p_ref[pl.ds(h * 128, 128), :, :] = word.reshape(128, V // 256, 128)


def _row_logsumexp(table):
    logz, packed = pl.pallas_call(
        _lse_body,
        grid=(_N_BLKS,),
        in_specs=[pl.BlockSpec((_ROWS_PER_BLK, V), lambda i: (i, 0))],
        out_specs=[
            pl.BlockSpec((1, 1, _ROWS_PER_BLK), lambda i: (i, 0, 0)),
            pl.BlockSpec((_ROWS_PER_BLK, V // 256, 128), lambda i: (i, 0, 0)),
        ],
        out_shape=[
            jax.ShapeDtypeStruct((_N_BLKS, 1, _ROWS_PER_BLK), jnp.float32),
            jax.ShapeDtypeStruct((V, V // 256, 128), jnp.int32),
        ],
    )(table)
    return logz.reshape(V), packed.reshape(V * V // 2)


_NC, _NS, _L = 2, 16, 16
_NW = _NC * _NS
_BPW = N_TOK // _NW
_CH = 128
_NCH = _BPW // _CH
_VPC = _CH // _L

_sc_mesh = plsc.VectorSubcoreMesh(core_axis_name="c", subcore_axis_name="s")


@functools.partial(
    pl.kernel,
    out_type=jax.ShapeDtypeStruct((_NW * _L,), jnp.float32),
    mesh=_sc_mesh,
    scratch_types=[
        pltpu.VMEM((_NCH, _CH), jnp.int32),
        pltpu.VMEM((_NCH, _CH), jnp.int32),
        pltpu.VMEM((_NCH, _CH), jnp.int32),
        pltpu.VMEM((_BPW,), jnp.int32),
        pltpu.VMEM((_BPW,), jnp.float32),
        pltpu.VMEM((_L,), jnp.float32),
        pltpu.SemaphoreType.DMA,
    ],
)
def _sc_gather(blocks_hbm, targets_hbm, packed_hbm, logz_hbm,
               out_hbm, idx_v, blk_v, tgt_v, vals_v, lz_v, part_v, sem):
    wid = lax.axis_index("s") * _NC + lax.axis_index("c")

    pltpu.sync_copy(blocks_hbm.at[wid], blk_v)
    pltpu.sync_copy(targets_hbm.at[wid], tgt_v)
    for j in range(_NCH):
        for k in range(_VPC):
            bb = blk_v[j, pl.ds(k * _L, _L)]
            tt = tgt_v[j, pl.ds(k * _L, _L)]
            idx_v[j, pl.ds(k * _L, _L)] = bb * (V // 2) + (tt & (V // 2 - 1))

    copies = []
    for j in range(_NCH):
        copies.append(
            pltpu.async_copy(packed_hbm.at[idx_v.at[j]],
                             vals_v.at[pl.ds(j * _CH, _CH)], sem))
        copies.append(
            pltpu.async_copy(logz_hbm.at[blk_v.at[j]],
                             lz_v.at[pl.ds(j * _CH, _CH)], sem))
    for cp in copies:
        cp.wait()

    acc = jnp.zeros((_L,), jnp.float32)
    for j in range(_NCH):
        for k in range(_VPC):
            off = j * _CH + k * _L
            w = vals_v[pl.ds(off, _L)]
            p = lax.shift_right_logical(tgt_v[j, pl.ds(k * _L, _L)], 12)
            bits = lax.shift_right_logical(w, p * 16) & 0xFFFF
            s = lax.shift_right_logical(bits, 15)
            e = lax.shift_right_logical(bits, 7) & 0xFF
            m = bits & 0x7F
            sign = 1.0 - 2.0 * s.astype(jnp.float32)
            frac = 1.0 + m.astype(jnp.float32) * (1.0 / 128.0)
            mag = jnp.exp((e.astype(jnp.float32) - 127.0)
                          * 0.6931471805599453)
            tv = sign * frac * mag
            lz = lz_v[pl.ds(off, _L)]
            acc = acc + (lz - tv)

    part_v[...] = acc
    pltpu.sync_copy(part_v, out_hbm.at[pl.ds(wid * _L, _L)])


def kernel(blocks, targets, table):
    b_sh = blocks.astype(jnp.int32).reshape(_NW, _NCH, _CH)
    t_sh = targets.astype(jnp.int32).reshape(_NW, _NCH, _CH)
    logz, packed = _row_logsumexp(table)
    parts = _sc_gather(b_sh, t_sh, packed, logz)
    return jnp.sum(parts) / N_TOK


# speedup vs baseline: 1.1881x; 1.0037x over previous
import functools

import jax
import jax.numpy as jnp
from jax import lax
from jax.experimental import pallas as pl
from jax.experimental.pallas import tpu as pltpu
from jax.experimental.pallas import tpu_sc as plsc

V = 8192
N_TOK = 256 * 128


_ROWS_PER_BLK = 512
_N_BLKS = V // _ROWS_PER_BLK


def _lse_body(x_ref, o_ref, p_ref):
    for h in range(_ROWS_PER_BLK // 64):
        x = x_ref[pl.ds(h * 64, 64), :]
        m = jnp.max(x, axis=1)
        s = jnp.sum(jnp.exp(x - m[:, None]), axis=1)
        o_ref[0, 0, pl.ds(h * 64, 64)] = m + jnp.log(s)
        u = lax.bitcast_convert_type(x, jnp.int32) + 0x8000
        word = (lax.shift_right_logical(u[:, : V // 2], 16)
                | (u[:, V // 2 :] & -0x10000))
        p_ref[pl.ds(h * 64, 64), :, :] = word.reshape(64, V // 256, 128)


def _row_logsumexp(table):
    logz, packed = pl.pallas_call(
        _lse_body,
        grid=(_N_BLKS,),
        in_specs=[pl.BlockSpec((_ROWS_PER_BLK, V), lambda i: (i, 0))],
        out_specs=[
            pl.BlockSpec((1, 1, _ROWS_PER_BLK), lambda i: (i, 0, 0)),
            pl.BlockSpec((_ROWS_PER_BLK, V // 256, 128), lambda i: (i, 0, 0)),
        ],
        out_shape=[
            jax.ShapeDtypeStruct((_N_BLKS, 1, _ROWS_PER_BLK), jnp.float32),
            jax.ShapeDtypeStruct((V, V // 256, 128), jnp.int32),
        ],
    )(table)
    return logz.reshape(V), packed.reshape(V * V // 2)


_NC, _NS, _L = 2, 16, 16
_NW = _NC * _NS
_BPW = N_TOK // _NW
_CH = 128
_NCH = _BPW // _CH
_VPC = _CH // _L

_sc_mesh = plsc.VectorSubcoreMesh(core_axis_name="c", subcore_axis_name="s")


@functools.partial(
    pl.kernel,
    out_type=jax.ShapeDtypeStruct((_NW * _L,), jnp.float32),
    mesh=_sc_mesh,
    scratch_types=[
        pltpu.VMEM((_NCH, _CH), jnp.int32),
        pltpu.VMEM((_NCH, _CH), jnp.int32),
        pltpu.VMEM((_NCH, _CH), jnp.int32),
        pltpu.VMEM((_BPW,), jnp.int32),
        pltpu.VMEM((_BPW,), jnp.float32),
        pltpu.VMEM((_L,), jnp.float32),
        pltpu.SemaphoreType.DMA,
    ],
)
def _sc_gather(blocks_hbm, targets_hbm, packed_hbm, logz_hbm,
               out_hbm, idx_v, blk_v, tgt_v, vals_v, lz_v, part_v, sem):
    wid = lax.axis_index("s") * _NC + lax.axis_index("c")

    pltpu.sync_copy(blocks_hbm.at[wid], blk_v)
    pltpu.sync_copy(targets_hbm.at[wid], tgt_v)
    for j in range(_NCH):
        for k in range(_VPC):
            bb = blk_v[j, pl.ds(k * _L, _L)]
            tt = tgt_v[j, pl.ds(k * _L, _L)]
            idx_v[j, pl.ds(k * _L, _L)] = bb * (V // 2) + (tt & (V // 2 - 1))

    copies = []
    for j in range(_NCH):
        copies.append(
            pltpu.async_copy(packed_hbm.at[idx_v.at[j]],
                             vals_v.at[pl.ds(j * _CH, _CH)], sem))
        copies.append(
            pltpu.async_copy(logz_hbm.at[blk_v.at[j]],
                             lz_v.at[pl.ds(j * _CH, _CH)], sem))
    for cp in copies:
        cp.wait()

    acc = jnp.zeros((_L,), jnp.float32)
    for j in range(_NCH):
        for k in range(_VPC):
            off = j * _CH + k * _L
            w = vals_v[pl.ds(off, _L)]
            p = lax.shift_right_logical(tgt_v[j, pl.ds(k * _L, _L)], 12)
            bits = lax.shift_right_logical(w, p * 16) & 0xFFFF
            s = lax.shift_right_logical(bits, 15)
            e = lax.shift_right_logical(bits, 7) & 0xFF
            m = bits & 0x7F
            sign = 1.0 - 2.0 * s.astype(jnp.float32)
            frac = 1.0 + m.astype(jnp.float32) * (1.0 / 128.0)
            mag = jnp.exp((e.astype(jnp.float32) - 127.0)
                          * 0.6931471805599453)
            tv = sign * frac * mag
            lz = lz_v[pl.ds(off, _L)]
            acc = acc + (lz - tv)

    part_v[...] = acc
    pltpu.sync_copy(part_v, out_hbm.at[pl.ds(wid * _L, _L)])


def kernel(blocks, targets, table):
    b_sh = blocks.astype(jnp.int32).reshape(_NW, _NCH, _CH)
    t_sh = targets.astype(jnp.int32).reshape(_NW, _NCH, _CH)
    logz, packed = _row_logsumexp(table)
    parts = _sc_gather(b_sh, t_sh, packed, logz)
    return jnp.sum(parts) / N_TOK
